# Initial kernel scaffold; baseline (speedup 1.0000x reference)
import jax
import jax.numpy as jnp
from jax.experimental import pallas as pl


def kernel(signal, codebook):
    raise NotImplementedError("write your pallas kernel here")


# speedup vs baseline: 2.7629x; 2.7629x over previous
import jax
import jax.numpy as jnp
from jax import lax
from jax.experimental import pallas as pl

_BLK = 2048


def _body(sig_ref, cb_ref, out_ref):
    x = sig_ref[0]
    cb = cb_ref[...]
    x2 = jnp.sum(x * x, axis=1, keepdims=True)
    c2 = jnp.sum(cb * cb, axis=1)
    cross = jax.lax.dot_general(
        x, cb, (((1,), (1,)), ((), ())),
        preferred_element_type=jnp.float32)
    d2 = x2 + c2[None, :] - 2.0 * cross
    md = jnp.min(d2, axis=1, keepdims=True)
    iota = lax.broadcasted_iota(jnp.int32, d2.shape, 1)
    first = jnp.min(jnp.where(d2 == md, iota, 32), axis=1, keepdims=True)
    onehot = (iota == first).astype(jnp.float32)
    out_ref[0] = jax.lax.dot_general(
        onehot, cb, (((1,), (0,)), ((), ())),
        preferred_element_type=jnp.float32)


def kernel(signal, codebook):
    b, n, d = signal.shape
    k = codebook.shape[0]
    grid = (b, n // _BLK)
    return pl.pallas_call(
        _body,
        grid=grid,
        in_specs=[
            pl.BlockSpec((1, _BLK, d), lambda i, j: (i, j, 0)),
            pl.BlockSpec((k, d), lambda i, j: (0, 0)),
        ],
        out_specs=pl.BlockSpec((1, _BLK, d), lambda i, j: (i, j, 0)),
        out_shape=jax.ShapeDtypeStruct((b, n, d), jnp.float32),
    )(signal, codebook)


# speedup vs baseline: 3.3619x; 1.2168x over previous
import jax
import jax.numpy as jnp
from jax import lax
from jax.experimental import pallas as pl

_BLK = 2048


def _body(sig_ref, cb_ref, out_ref):
    x = sig_ref[0]
    cb = cb_ref[...]
    c2 = jnp.sum(cb * cb, axis=1)
    cross_t = lax.dot_general(
        cb, x, (((1,), (1,)), ((), ())),
        preferred_element_type=jnp.float32)
    s = c2[:, None] - 2.0 * cross_t
    md = jnp.min(s, axis=0, keepdims=True)
    iota = lax.broadcasted_iota(jnp.int32, s.shape, 0)
    first = jnp.min(jnp.where(s == md, iota, 32), axis=0, keepdims=True)
    onehot = (iota == first).astype(jnp.float32)
    out_t = lax.dot_general(
        cb, onehot, (((0,), (0,)), ((), ())),
        preferred_element_type=jnp.float32)
    out_ref[0] = out_t.T


def kernel(signal, codebook):
    b, n, d = signal.shape
    k = codebook.shape[0]
    grid = (b, n // _BLK)
    return pl.pallas_call(
        _body,
        grid=grid,
        in_specs=[
            pl.BlockSpec((1, _BLK, d), lambda i, j: (i, j, 0)),
            pl.BlockSpec((k, d), lambda i, j: (0, 0)),
        ],
        out_specs=pl.BlockSpec((1, _BLK, d), lambda i, j: (i, j, 0)),
        out_shape=jax.ShapeDtypeStruct((b, n, d), jnp.float32),
    )(signal, codebook)


# speedup vs baseline: 9.8955x; 2.9434x over previous
import jax
import jax.numpy as jnp
from jax import lax
from jax.experimental import pallas as pl

_BLKN = 4096


def _body(sig_ref, cb_ref, out_ref):
    x_t = sig_ref[0]
    cb = cb_ref[...]
    c2 = jnp.sum(cb * cb, axis=1)
    cross_t = lax.dot_general(
        cb, x_t, (((1,), (0,)), ((), ())),
        preferred_element_type=jnp.float32)
    s = c2[:, None] - 2.0 * cross_t
    md = jnp.min(s, axis=0, keepdims=True)
    iota = lax.broadcasted_iota(jnp.int32, s.shape, 0)
    first = jnp.min(jnp.where(s == md, iota, 32), axis=0, keepdims=True)
    onehot = (iota == first).astype(jnp.float32)
    out_ref[0] = lax.dot_general(
        cb, onehot, (((0,), (0,)), ((), ())),
        preferred_element_type=jnp.float32)


def kernel(signal, codebook):
    b, n, d = signal.shape
    k = codebook.shape[0]
    sig_t = jnp.transpose(signal, (0, 2, 1))
    grid = (b, n // _BLKN)
    out_t = pl.pallas_call(
        _body,
        grid=grid,
        in_specs=[
            pl.BlockSpec((1, d, _BLKN), lambda i, j: (i, 0, j)),
            pl.BlockSpec((k, d), lambda i, j: (0, 0)),
        ],
        out_specs=pl.BlockSpec((1, d, _BLKN), lambda i, j: (i, 0, j)),
        out_shape=jax.ShapeDtypeStruct((b, d, n), jnp.float32),
    )(sig_t, codebook)
    return jnp.transpose(out_t, (0, 2, 1))


# speedup vs baseline: 13.5453x; 1.3688x over previous
import jax
import jax.numpy as jnp
from jax import lax
from jax.experimental import pallas as pl

_BLKN = 16384


def _body(sig_ref, cb_ref, out_ref):
    x_t = sig_ref[0]
    cb = cb_ref[...]
    c2 = jnp.sum(cb * cb, axis=1)
    cross_t = lax.dot_general(
        cb, x_t, (((1,), (0,)), ((), ())),
        preferred_element_type=jnp.float32)
    s = c2[:, None] - 2.0 * cross_t
    md = jnp.min(s, axis=0, keepdims=True)
    iota = lax.broadcasted_iota(jnp.int32, s.shape, 0)
    first = jnp.min(jnp.where(s == md, iota, 32), axis=0, keepdims=True)
    onehot = (iota == first).astype(jnp.float32)
    out_ref[0] = lax.dot_general(
        cb, onehot, (((0,), (0,)), ((), ())),
        preferred_element_type=jnp.float32)


def kernel(signal, codebook):
    b, n, d = signal.shape
    k = codebook.shape[0]
    sig_t = jnp.transpose(signal, (0, 2, 1))
    grid = (b, n // _BLKN)
    out_t = pl.pallas_call(
        _body,
        grid=grid,
        in_specs=[
            pl.BlockSpec((1, d, _BLKN), lambda i, j: (i, 0, j)),
            pl.BlockSpec((k, d), lambda i, j: (0, 0)),
        ],
        out_specs=pl.BlockSpec((1, d, _BLKN), lambda i, j: (i, 0, j)),
        out_shape=jax.ShapeDtypeStruct((b, d, n), jnp.float32),
    )(sig_t, codebook)
    return jnp.transpose(out_t, (0, 2, 1))
